# baseline (speedup 1.0000x reference)
import functools

import jax
import jax.numpy as jnp
from jax import lax
from jax.experimental import pallas as pl
from jax.experimental.pallas import tpu as pltpu
from jax.experimental.pallas import tpu_sc as plsc

NUM_FIELDS = 26
BATCH = 4096
DIM = 32


@functools.cache
def _build():
    info = plsc.get_sparse_core_info()
    nc, ns = info.num_cores, info.num_subcores
    nw = nc * ns
    b_per_w = BATCH // nw

    mesh = plsc.VectorSubcoreMesh(core_axis_name="c", subcore_axis_name="s")
    out_type = tuple(
        jax.ShapeDtypeStruct((BATCH, DIM), jnp.float32) for _ in range(NUM_FIELDS)
    )

    @functools.partial(
        pl.kernel,
        out_type=out_type,
        mesh=mesh,
        scratch_types=[
            pltpu.VMEM((NUM_FIELDS, b_per_w), jnp.int32),
            pltpu.VMEM((NUM_FIELDS, b_per_w, DIM), jnp.float32),
            pltpu.SemaphoreType.DMA,
        ],
        compiler_params=pltpu.CompilerParams(use_tc_tiling_on_sc=False),
    )
    def k(*refs):
        idx_refs = refs[:NUM_FIELDS]
        table_refs = refs[NUM_FIELDS : 2 * NUM_FIELDS]
        out_refs = refs[2 * NUM_FIELDS : 3 * NUM_FIELDS]
        idx_v, rows_v, sem = refs[3 * NUM_FIELDS :]

        wid = lax.axis_index("s") * nc + lax.axis_index("c")
        base = wid * b_per_w

        for i in range(NUM_FIELDS):
            pltpu.sync_copy(idx_refs[i].at[pl.ds(base, b_per_w)], idx_v.at[i])
        copies = [
            pltpu.async_copy(table_refs[i].at[idx_v.at[i]], rows_v.at[i], sem)
            for i in range(NUM_FIELDS)
        ]
        for i in range(NUM_FIELDS):
            copies[i].wait()
            pltpu.sync_copy(rows_v.at[i], out_refs[i].at[pl.ds(base, b_per_w)])

    return k


def kernel(*args):
    return _build()(*args)


# speedup vs baseline: 1.2389x; 1.2389x over previous
import functools

import jax
import jax.numpy as jnp
from jax import lax
from jax.experimental import pallas as pl
from jax.experimental.pallas import tpu as pltpu
from jax.experimental.pallas import tpu_sc as plsc

NUM_FIELDS = 26
BATCH = 4096
DIM = 32


@functools.cache
def _build():
    info = plsc.get_sparse_core_info()
    nc, ns, lanes = info.num_cores, info.num_subcores, info.num_lanes
    nw = nc * ns
    bw = BATCH // nw

    mesh = plsc.VectorSubcoreMesh(core_axis_name="c", subcore_axis_name="s")
    out_type = tuple(
        jax.ShapeDtypeStruct((BATCH, DIM), jnp.float32) for _ in range(NUM_FIELDS)
    )

    @functools.partial(
        pl.kernel,
        out_type=out_type,
        mesh=mesh,
        scratch_types=[
            pltpu.VMEM((2, bw), jnp.int32),
            pltpu.VMEM((bw, DIM), jnp.float32),
            pltpu.VMEM((bw, DIM), jnp.float32),
            pltpu.SemaphoreType.DMA,
            pltpu.SemaphoreType.DMA,
            pltpu.SemaphoreType.DMA,
            pltpu.SemaphoreType.DMA,
        ],
    )
    def k(*refs):
        idx_refs = refs[:NUM_FIELDS]
        tbl_refs = refs[NUM_FIELDS : 2 * NUM_FIELDS]
        out_refs = refs[2 * NUM_FIELDS : 3 * NUM_FIELDS]
        idx_v, rows0, rows1, sem_i, sem_g, sem_s0, sem_s1 = refs[
            3 * NUM_FIELDS :
        ]
        rows = (rows0, rows1)
        sem_s = (sem_s0, sem_s1)

        wid = lax.axis_index("s") * nc + lax.axis_index("c")
        base = wid * bw

        def drain_idx(p):
            pltpu.make_async_copy(
                idx_refs[0].at[pl.ds(base, bw)], idx_v.at[p], sem_i
            ).wait()

        pltpu.async_copy(idx_refs[0].at[pl.ds(base, bw)], idx_v.at[0], sem_i)
        drain_idx(0)

        for f in range(NUM_FIELDS):
            p = f % 2
            tbl = tbl_refs[f]
            rows_v = rows[p]

            if f + 1 < NUM_FIELDS:
                pltpu.async_copy(
                    idx_refs[f + 1].at[pl.ds(base, bw)], idx_v.at[1 - p], sem_i
                )

            if f >= 2:
                pltpu.make_async_copy(
                    rows_v, out_refs[f - 2].at[pl.ds(base, bw)], sem_s[p]
                ).wait()

            def fire_block(ch, _, tbl=tbl, rows_v=rows_v, p=p):
                v = idx_v[p, pl.ds(ch * lanes, lanes)]
                for j in range(lanes):
                    r = v[j]
                    pltpu.async_copy(
                        tbl.at[pl.ds(r, 1)],
                        rows_v.at[pl.ds(ch * lanes + j, 1)],
                        sem_g,
                    )
                return 0

            lax.fori_loop(0, bw // lanes, fire_block, 0)

            pltpu.make_async_copy(
                tbl.at[pl.ds(0, bw)], rows_v, sem_g
            ).wait()

            pltpu.async_copy(rows_v, out_refs[f].at[pl.ds(base, bw)], sem_s[p])

            if f + 1 < NUM_FIELDS:
                drain_idx(1 - p)

        for f in (NUM_FIELDS - 2, NUM_FIELDS - 1):
            p = f % 2
            pltpu.make_async_copy(
                rows[p], out_refs[f].at[pl.ds(base, bw)], sem_s[p]
            ).wait()

    return k


def kernel(*args):
    return _build()(*args)
